# baseline (speedup 1.0000x reference)
import functools

import jax
import jax.numpy as jnp
from jax import lax
from jax.experimental import pallas as pl
from jax.experimental.pallas import tpu as pltpu
from jax.experimental.pallas import tpu_sc as plsc

N = 10000
E = 320000
D = 128
K = 5
C = 64

BR = 400
NB = N // BR
NC, NS = 2, 16
NW = NC * NS
B = 128
NFULL = 80
NIH = NFULL // 2
EPW = NFULL * B
EPAD = NW * EPW
NROWS = N + 16
RPT = 624
ZR = 80


def _mlp_body(x_ref, w_ref, b_ref, o_ref):
    o_ref[...] = jnp.maximum(
        jnp.dot(x_ref[...], w_ref[...], preferred_element_type=jnp.float32)
        + b_ref[...], 0.0)


_mlp = pl.pallas_call(
    _mlp_body,
    grid=(NB,),
    in_specs=[pl.BlockSpec((BR, D), lambda i: (i, 0)),
              pl.BlockSpec((D, D), lambda i: (0, 0)),
              pl.BlockSpec((1, D), lambda i: (0, 0))],
    out_specs=pl.BlockSpec((BR, D), lambda i: (i, 0)),
    out_shape=jax.ShapeDtypeStruct((N, D), jnp.float32),
)


def _combine_body(h_ref, a0_ref, a1_ref, w_ref, b_ref, o_ref):
    s = h_ref[...] + a0_ref[...] + a1_ref[...]
    o_ref[...] = jnp.maximum(
        jnp.dot(s, w_ref[...], preferred_element_type=jnp.float32)
        + b_ref[...], 0.0)


_combine = pl.pallas_call(
    _combine_body,
    grid=(NB,),
    in_specs=[pl.BlockSpec((BR, D), lambda i: (i, 0)),
              pl.BlockSpec((BR, D), lambda i: (i, 0)),
              pl.BlockSpec((BR, D), lambda i: (i, 0)),
              pl.BlockSpec((D, D), lambda i: (0, 0)),
              pl.BlockSpec((1, D), lambda i: (0, 0))],
    out_specs=pl.BlockSpec((BR, D), lambda i: (i, 0)),
    out_shape=jax.ShapeDtypeStruct((N, D), jnp.float32),
)


def _scale_body(hop_ref, h_ref, o_ref):
    hrow = hop_ref[...]
    m = jnp.max(hrow)
    e = jnp.exp(hrow - m)
    w = e / jnp.sum(e)
    hb = h_ref[...]
    for kk in range(K):
        o_ref[kk] = hb * w[0, kk]


_scale = pl.pallas_call(
    _scale_body,
    grid=(NB,),
    in_specs=[pl.BlockSpec((1, K), lambda i: (0, 0)),
              pl.BlockSpec((BR, D), lambda i: (i, 0))],
    out_specs=pl.BlockSpec((K, BR, D), lambda i: (0, i, 0)),
    out_shape=jax.ShapeDtypeStruct((K, N, D), jnp.float32),
)


def _head_body(h_ref, w1_ref, b1_ref, w2_ref, b2_ref, o_ref):
    t = jnp.maximum(
        jnp.dot(h_ref[...], w1_ref[...], preferred_element_type=jnp.float32)
        + b1_ref[...], 0.0)
    o_ref[...] = (jnp.dot(t, w2_ref[...], preferred_element_type=jnp.float32)
                  + b2_ref[...])


_head = pl.pallas_call(
    _head_body,
    grid=(NB,),
    in_specs=[pl.BlockSpec((BR, D), lambda i: (i, 0)),
              pl.BlockSpec((D, D), lambda i: (0, 0)),
              pl.BlockSpec((1, D), lambda i: (0, 0)),
              pl.BlockSpec((D, C), lambda i: (0, 0)),
              pl.BlockSpec((1, C), lambda i: (0, 0))],
    out_specs=pl.BlockSpec((BR, C), lambda i: (i, 0)),
    out_shape=jax.ShapeDtypeStruct((N, C), jnp.float32),
)


def _gidx_body(src_ref, ew_ref, o_ref):
    o_ref[...] = ew_ref[...] * N + src_ref[...]


_gidx = pl.pallas_call(
    _gidx_body,
    out_shape=jax.ShapeDtypeStruct((EPAD // 128, 128), jnp.int32),
)


_mesh = plsc.VectorSubcoreMesh(core_axis_name="c", subcore_axis_name="s")


@functools.partial(
    pl.kernel,
    out_type=jax.ShapeDtypeStruct((NC, N, D), jnp.float32),
    mesh=_mesh,
    scratch_types=[
        pltpu.VMEM((B,), jnp.int32),
        pltpu.VMEM((B,), jnp.int32),
        pltpu.VMEM((B,), jnp.int32),
        pltpu.VMEM((B,), jnp.int32),
        pltpu.VMEM((B, D), jnp.float32),
        pltpu.VMEM((B, D), jnp.float32),
        pltpu.VMEM((ZR, D), jnp.float32),
        pltpu.VMEM_SHARED((NROWS, D), jnp.float32),
        pltpu.SemaphoreType.DMA,
        pltpu.SemaphoreType.DMA,
    ],
)
def _sc_agg(scaled_hbm, gidx_hbm, dst_hbm, out_hbm,
            gi0_v, di0_v, gi1_v, di1_v, rows0_v, rows1_v, zbuf_v, acc_sh,
            sem0, sem1):
    cid = lax.axis_index("c")
    sid = lax.axis_index("s")
    wid = cid * NS + sid

    zv = jnp.zeros((16,), jnp.float32)

    def _zb(i, carry):
        zbuf_v[i // 8, pl.ds((i % 8) * 16, 16)] = zv
        return carry

    lax.fori_loop(0, ZR * 8, _zb, 0)
    r0 = sid * RPT
    nz = RPT // ZR
    for j in range(nz):
        pltpu.sync_copy(zbuf_v, acc_sh.at[pl.ds(r0 + j * ZR, ZR)])
    rem = RPT - nz * ZR
    pltpu.sync_copy(zbuf_v.at[pl.ds(0, rem)],
                    acc_sh.at[pl.ds(r0 + nz * ZR, rem)])

    @pl.when(sid == 0)
    def _zero_tail():
        pltpu.sync_copy(zbuf_v.at[pl.ds(0, 16)],
                        acc_sh.at[pl.ds(NS * RPT, 16)])

    plsc.subcore_barrier()

    base = wid * EPW

    def _pair(i, carry):
        off0 = base + (2 * i) * B
        off1 = off0 + B
        pltpu.sync_copy(gidx_hbm.at[pl.ds(off0, B)], gi0_v)
        pltpu.sync_copy(dst_hbm.at[pl.ds(off0, B)], di0_v)
        g0 = pltpu.async_copy(scaled_hbm.at[gi0_v], rows0_v, sem0)
        pltpu.sync_copy(gidx_hbm.at[pl.ds(off1, B)], gi1_v)
        pltpu.sync_copy(dst_hbm.at[pl.ds(off1, B)], di1_v)
        g1 = pltpu.async_copy(scaled_hbm.at[gi1_v], rows1_v, sem1)
        g0.wait()
        pltpu.sync_copy(rows0_v, acc_sh.at[di0_v], add=True)
        g1.wait()
        pltpu.sync_copy(rows1_v, acc_sh.at[di1_v], add=True)
        return carry

    lax.fori_loop(0, NIH, _pair, 0)

    plsc.subcore_barrier()
    pltpu.sync_copy(acc_sh.at[pl.ds(r0, RPT)],
                    out_hbm.at[cid, pl.ds(r0, RPT)])

    @pl.when(sid == 0)
    def _flush_tail():
        pltpu.sync_copy(acc_sh.at[pl.ds(NS * RPT, 16)],
                        out_hbm.at[cid, pl.ds(NS * RPT, 16)])


def kernel(x, edge_index, edge_weights, W0, b0, hop1, W1, b1,
           hop2, W2, b2, Wh1, bh1, Wh2, bh2):
    src = edge_index[0]
    dst = edge_index[1]
    pad = EPAD - E
    srcp = jnp.concatenate([src, jnp.zeros((pad,), jnp.int32)])
    ewp = jnp.concatenate([edge_weights, jnp.zeros((pad,), jnp.int32)])
    dstp = jnp.concatenate([dst, jnp.full((pad,), N, jnp.int32)])
    gidx = _gidx(srcp.reshape(EPAD // 128, 128),
                 ewp.reshape(EPAD // 128, 128)).reshape(EPAD)

    b0r = b0.reshape(1, D)
    h = _mlp(x, W0, b0r)
    for hop, W, b in ((hop1, W1, b1), (hop2, W2, b2)):
        s = _scale(hop.reshape(1, K), h).reshape(K * N, D)
        p = _sc_agg(s, gidx, dstp)
        h = _combine(h, p[0], p[1], W, b.reshape(1, D))
    out = _head(h, Wh1, bh1.reshape(1, D), Wh2, bh2.reshape(1, C))
    return out


# speedup vs baseline: 2.5383x; 2.5383x over previous
import functools

import jax
import jax.numpy as jnp
from jax import lax
from jax.experimental import pallas as pl
from jax.experimental.pallas import tpu as pltpu
from jax.experimental.pallas import tpu_sc as plsc

N = 10000
E = 320000
D = 128
K = 5
C = 64

BR = 400
NB = N // BR
NC, NS = 2, 16
NW = NC * NS
B = 128
NCHUNK = E // B
NCW = NCHUNK // NW
NCX = NCHUNK % NW
NIH = NCW // 2
NROWS = N
RPT = 624
ZR = 80


def _mlp_body(x_ref, w_ref, b_ref, o_ref):
    o_ref[...] = jnp.maximum(
        jnp.dot(x_ref[...], w_ref[...], preferred_element_type=jnp.float32)
        + b_ref[...], 0.0)


_mlp = pl.pallas_call(
    _mlp_body,
    grid=(NB,),
    in_specs=[pl.BlockSpec((BR, D), lambda i: (i, 0)),
              pl.BlockSpec((D, D), lambda i: (0, 0)),
              pl.BlockSpec((1, D), lambda i: (0, 0))],
    out_specs=pl.BlockSpec((BR, D), lambda i: (i, 0)),
    out_shape=jax.ShapeDtypeStruct((N, D), jnp.float32),
)


def _combine_body(h_ref, a0_ref, a1_ref, w_ref, b_ref, o_ref):
    s = h_ref[...] + a0_ref[...] + a1_ref[...]
    o_ref[...] = jnp.maximum(
        jnp.dot(s, w_ref[...], preferred_element_type=jnp.float32)
        + b_ref[...], 0.0)


_combine = pl.pallas_call(
    _combine_body,
    grid=(NB,),
    in_specs=[pl.BlockSpec((BR, D), lambda i: (i, 0)),
              pl.BlockSpec((BR, D), lambda i: (i, 0)),
              pl.BlockSpec((BR, D), lambda i: (i, 0)),
              pl.BlockSpec((D, D), lambda i: (0, 0)),
              pl.BlockSpec((1, D), lambda i: (0, 0))],
    out_specs=pl.BlockSpec((BR, D), lambda i: (i, 0)),
    out_shape=jax.ShapeDtypeStruct((N, D), jnp.float32),
)


def _scale_body(hop_ref, h_ref, o_ref):
    hrow = hop_ref[...]
    m = jnp.max(hrow)
    e = jnp.exp(hrow - m)
    w = e / jnp.sum(e)
    hb = h_ref[...]
    for kk in range(K):
        o_ref[kk] = hb * w[0, kk]


_scale = pl.pallas_call(
    _scale_body,
    grid=(NB,),
    in_specs=[pl.BlockSpec((1, K), lambda i: (0, 0)),
              pl.BlockSpec((BR, D), lambda i: (i, 0))],
    out_specs=pl.BlockSpec((K, BR, D), lambda i: (0, i, 0)),
    out_shape=jax.ShapeDtypeStruct((K, N, D), jnp.float32),
)


def _head_body(h_ref, w1_ref, b1_ref, w2_ref, b2_ref, o_ref):
    t = jnp.maximum(
        jnp.dot(h_ref[...], w1_ref[...], preferred_element_type=jnp.float32)
        + b1_ref[...], 0.0)
    o_ref[...] = (jnp.dot(t, w2_ref[...], preferred_element_type=jnp.float32)
                  + b2_ref[...])


_head = pl.pallas_call(
    _head_body,
    grid=(NB,),
    in_specs=[pl.BlockSpec((BR, D), lambda i: (i, 0)),
              pl.BlockSpec((D, D), lambda i: (0, 0)),
              pl.BlockSpec((1, D), lambda i: (0, 0)),
              pl.BlockSpec((D, C), lambda i: (0, 0)),
              pl.BlockSpec((1, C), lambda i: (0, 0))],
    out_specs=pl.BlockSpec((BR, C), lambda i: (i, 0)),
    out_shape=jax.ShapeDtypeStruct((N, C), jnp.float32),
)


def _gidx_body(src_ref, ew_ref, o_ref):
    o_ref[...] = ew_ref[...] * N + src_ref[...]


_gidx = pl.pallas_call(
    _gidx_body,
    out_shape=jax.ShapeDtypeStruct((NCHUNK, B), jnp.int32),
)


_mesh = plsc.VectorSubcoreMesh(core_axis_name="c", subcore_axis_name="s")


@functools.partial(
    pl.kernel,
    out_type=jax.ShapeDtypeStruct((NC, N, D), jnp.float32),
    mesh=_mesh,
    scratch_types=[
        pltpu.VMEM((B,), jnp.int32),
        pltpu.VMEM((B,), jnp.int32),
        pltpu.VMEM((B,), jnp.int32),
        pltpu.VMEM((B,), jnp.int32),
        pltpu.VMEM((B, D), jnp.float32),
        pltpu.VMEM((B, D), jnp.float32),
        pltpu.VMEM((ZR, D), jnp.float32),
        pltpu.VMEM_SHARED((NROWS, D), jnp.float32),
        pltpu.SemaphoreType.DMA,
        pltpu.SemaphoreType.DMA,
    ],
)
def _sc_agg(scaled_hbm, gidx_hbm, dst_hbm, out_hbm,
            gi0_v, di0_v, gi1_v, di1_v, rows0_v, rows1_v, zbuf_v, acc_sh,
            sem0, sem1):
    cid = lax.axis_index("c")
    sid = lax.axis_index("s")
    wid = cid * NS + sid

    zv = jnp.zeros((16,), jnp.float32)

    def _zb(i, carry):
        zbuf_v[i // 8, pl.ds((i % 8) * 16, 16)] = zv
        return carry

    lax.fori_loop(0, ZR * 8, _zb, 0)
    r0 = sid * RPT
    nz = RPT // ZR
    for j in range(nz):
        pltpu.sync_copy(zbuf_v, acc_sh.at[pl.ds(r0 + j * ZR, ZR)])
    rem = RPT - nz * ZR
    pltpu.sync_copy(zbuf_v.at[pl.ds(0, rem)],
                    acc_sh.at[pl.ds(r0 + nz * ZR, rem)])

    @pl.when(sid == 0)
    def _zero_tail():
        pltpu.sync_copy(zbuf_v.at[pl.ds(0, 16)],
                        acc_sh.at[pl.ds(NS * RPT, 16)])

    plsc.subcore_barrier()

    base = (NCW * wid + jnp.minimum(wid, NCX)) * B

    def _do_chunk(off, gi_v, di_v, rows_v, sem):
        pltpu.sync_copy(gidx_hbm.at[pl.ds(off, B)], gi_v)
        pltpu.sync_copy(dst_hbm.at[pl.ds(off, B)], di_v)
        return pltpu.async_copy(scaled_hbm.at[gi_v], rows_v, sem)

    def _pair(i, carry):
        off0 = base + (2 * i) * B
        g0 = _do_chunk(off0, gi0_v, di0_v, rows0_v, sem0)
        g1 = _do_chunk(off0 + B, gi1_v, di1_v, rows1_v, sem1)
        g0.wait()
        pltpu.sync_copy(rows0_v, acc_sh.at[di0_v], add=True)
        g1.wait()
        pltpu.sync_copy(rows1_v, acc_sh.at[di1_v], add=True)
        return carry

    lax.fori_loop(0, NIH, _pair, 0)

    @pl.when(wid < NCX)
    def _extra_chunk():
        g = _do_chunk(base + NCW * B, gi0_v, di0_v, rows0_v, sem0)
        g.wait()
        pltpu.sync_copy(rows0_v, acc_sh.at[di0_v], add=True)

    plsc.subcore_barrier()
    pltpu.sync_copy(acc_sh.at[pl.ds(r0, RPT)],
                    out_hbm.at[cid, pl.ds(r0, RPT)])

    @pl.when(sid == 0)
    def _flush_tail():
        pltpu.sync_copy(acc_sh.at[pl.ds(NS * RPT, 16)],
                        out_hbm.at[cid, pl.ds(NS * RPT, 16)])


def kernel(x, edge_index, edge_weights, W0, b0, hop1, W1, b1,
           hop2, W2, b2, Wh1, bh1, Wh2, bh2):
    src = edge_index[0]
    dst = edge_index[1]
    gidx = _gidx(src.reshape(NCHUNK, B),
                 edge_weights.reshape(NCHUNK, B)).reshape(E)

    b0r = b0.reshape(1, D)
    h = _mlp(x, W0, b0r)
    for hop, W, b in ((hop1, W1, b1), (hop2, W2, b2)):
        s = _scale(hop.reshape(1, K), h).reshape(K * N, D)
        p = _sc_agg(s, gidx, dst)
        h = _combine(h, p[0], p[1], W, b.reshape(1, D))
    out = _head(h, Wh1, bh1.reshape(1, D), Wh2, bh2.reshape(1, C))
    return out


# speedup vs baseline: 2.7780x; 1.0945x over previous
import functools

import jax
import jax.numpy as jnp
from jax import lax
from jax.experimental import pallas as pl
from jax.experimental.pallas import tpu as pltpu
from jax.experimental.pallas import tpu_sc as plsc

N = 10000
E = 320000
D = 128
K = 5
C = 64

BR = 400
NB = N // BR
NC, NS = 2, 16
NW = NC * NS
B = 128
NCHUNK = E // B
NCW = NCHUNK // NW
NCX = NCHUNK % NW
NIH = NCW // 2
NROWS = N
RPT = 624
ZR = 80


def _softmax_row(hop_ref):
    hrow = hop_ref[...]
    m = jnp.max(hrow)
    e = jnp.exp(hrow - m)
    return e / jnp.sum(e)


def _mlp_scale_body(x_ref, w_ref, b_ref, hop_ref, h_ref, s_ref):
    h = jnp.maximum(
        jnp.dot(x_ref[...], w_ref[...], preferred_element_type=jnp.float32)
        + b_ref[...], 0.0)
    h_ref[...] = h
    w = _softmax_row(hop_ref)
    for kk in range(K):
        s_ref[kk] = h * w[0, kk]


_mlp_scale = pl.pallas_call(
    _mlp_scale_body,
    grid=(NB,),
    in_specs=[pl.BlockSpec((BR, D), lambda i: (i, 0)),
              pl.BlockSpec((D, D), lambda i: (0, 0)),
              pl.BlockSpec((1, D), lambda i: (0, 0)),
              pl.BlockSpec((1, K), lambda i: (0, 0))],
    out_specs=[pl.BlockSpec((BR, D), lambda i: (i, 0)),
               pl.BlockSpec((K, BR, D), lambda i: (0, i, 0))],
    out_shape=[jax.ShapeDtypeStruct((N, D), jnp.float32),
               jax.ShapeDtypeStruct((K, N, D), jnp.float32)],
)


def _combine_scale_body(h_ref, a0_ref, a1_ref, w_ref, b_ref, hop_ref,
                        h1_ref, s_ref):
    s = h_ref[...] + a0_ref[...] + a1_ref[...]
    h1 = jnp.maximum(
        jnp.dot(s, w_ref[...], preferred_element_type=jnp.float32)
        + b_ref[...], 0.0)
    h1_ref[...] = h1
    w = _softmax_row(hop_ref)
    for kk in range(K):
        s_ref[kk] = h1 * w[0, kk]


_combine_scale = pl.pallas_call(
    _combine_scale_body,
    grid=(NB,),
    in_specs=[pl.BlockSpec((BR, D), lambda i: (i, 0)),
              pl.BlockSpec((BR, D), lambda i: (i, 0)),
              pl.BlockSpec((BR, D), lambda i: (i, 0)),
              pl.BlockSpec((D, D), lambda i: (0, 0)),
              pl.BlockSpec((1, D), lambda i: (0, 0)),
              pl.BlockSpec((1, K), lambda i: (0, 0))],
    out_specs=[pl.BlockSpec((BR, D), lambda i: (i, 0)),
               pl.BlockSpec((K, BR, D), lambda i: (0, i, 0))],
    out_shape=[jax.ShapeDtypeStruct((N, D), jnp.float32),
               jax.ShapeDtypeStruct((K, N, D), jnp.float32)],
)


def _combine_head_body(h_ref, a0_ref, a1_ref, w_ref, b_ref,
                       w1_ref, b1_ref, w2_ref, b2_ref, o_ref):
    s = h_ref[...] + a0_ref[...] + a1_ref[...]
    h2 = jnp.maximum(
        jnp.dot(s, w_ref[...], preferred_element_type=jnp.float32)
        + b_ref[...], 0.0)
    t = jnp.maximum(
        jnp.dot(h2, w1_ref[...], preferred_element_type=jnp.float32)
        + b1_ref[...], 0.0)
    o_ref[...] = (jnp.dot(t, w2_ref[...], preferred_element_type=jnp.float32)
                  + b2_ref[...])


_combine_head = pl.pallas_call(
    _combine_head_body,
    grid=(NB,),
    in_specs=[pl.BlockSpec((BR, D), lambda i: (i, 0)),
              pl.BlockSpec((BR, D), lambda i: (i, 0)),
              pl.BlockSpec((BR, D), lambda i: (i, 0)),
              pl.BlockSpec((D, D), lambda i: (0, 0)),
              pl.BlockSpec((1, D), lambda i: (0, 0)),
              pl.BlockSpec((D, D), lambda i: (0, 0)),
              pl.BlockSpec((1, D), lambda i: (0, 0)),
              pl.BlockSpec((D, C), lambda i: (0, 0)),
              pl.BlockSpec((1, C), lambda i: (0, 0))],
    out_specs=pl.BlockSpec((BR, C), lambda i: (i, 0)),
    out_shape=jax.ShapeDtypeStruct((N, C), jnp.float32),
)


def _gidx_body(src_ref, ew_ref, o_ref):
    o_ref[...] = ew_ref[...] * N + src_ref[...]


_gidx = pl.pallas_call(
    _gidx_body,
    out_shape=jax.ShapeDtypeStruct((NCHUNK, B), jnp.int32),
)


_mesh = plsc.VectorSubcoreMesh(core_axis_name="c", subcore_axis_name="s")


@functools.partial(
    pl.kernel,
    out_type=jax.ShapeDtypeStruct((NC, N, D), jnp.float32),
    mesh=_mesh,
    scratch_types=[
        pltpu.VMEM((B,), jnp.int32),
        pltpu.VMEM((B,), jnp.int32),
        pltpu.VMEM((B,), jnp.int32),
        pltpu.VMEM((B,), jnp.int32),
        pltpu.VMEM((B, D), jnp.float32),
        pltpu.VMEM((B, D), jnp.float32),
        pltpu.VMEM((ZR, D), jnp.float32),
        pltpu.VMEM_SHARED((NROWS, D), jnp.float32),
        pltpu.SemaphoreType.DMA,
        pltpu.SemaphoreType.DMA,
    ],
)
def _sc_agg(scaled_hbm, gidx_hbm, dst_hbm, out_hbm,
            gi0_v, di0_v, gi1_v, di1_v, rows0_v, rows1_v, zbuf_v, acc_sh,
            sem0, sem1):
    cid = lax.axis_index("c")
    sid = lax.axis_index("s")
    wid = cid * NS + sid

    zv = jnp.zeros((16,), jnp.float32)

    def _zb(i, carry):
        zbuf_v[i // 8, pl.ds((i % 8) * 16, 16)] = zv
        return carry

    lax.fori_loop(0, ZR * 8, _zb, 0)
    r0 = sid * RPT
    nz = RPT // ZR
    for j in range(nz):
        pltpu.sync_copy(zbuf_v, acc_sh.at[pl.ds(r0 + j * ZR, ZR)])
    rem = RPT - nz * ZR
    pltpu.sync_copy(zbuf_v.at[pl.ds(0, rem)],
                    acc_sh.at[pl.ds(r0 + nz * ZR, rem)])

    @pl.when(sid == 0)
    def _zero_tail():
        pltpu.sync_copy(zbuf_v.at[pl.ds(0, 16)],
                        acc_sh.at[pl.ds(NS * RPT, 16)])

    plsc.subcore_barrier()

    base = (NCW * wid + jnp.minimum(wid, NCX)) * B

    def _do_chunk(off, gi_v, di_v, rows_v, sem):
        pltpu.sync_copy(gidx_hbm.at[pl.ds(off, B)], gi_v)
        pltpu.sync_copy(dst_hbm.at[pl.ds(off, B)], di_v)
        return pltpu.async_copy(scaled_hbm.at[gi_v], rows_v, sem)

    def _pair(i, carry):
        off0 = base + (2 * i) * B
        g0 = _do_chunk(off0, gi0_v, di0_v, rows0_v, sem0)
        g1 = _do_chunk(off0 + B, gi1_v, di1_v, rows1_v, sem1)
        g0.wait()
        pltpu.sync_copy(rows0_v, acc_sh.at[di0_v], add=True)
        g1.wait()
        pltpu.sync_copy(rows1_v, acc_sh.at[di1_v], add=True)
        return carry

    lax.fori_loop(0, NIH, _pair, 0)

    @pl.when(wid < NCX)
    def _extra_chunk():
        g = _do_chunk(base + NCW * B, gi0_v, di0_v, rows0_v, sem0)
        g.wait()
        pltpu.sync_copy(rows0_v, acc_sh.at[di0_v], add=True)

    plsc.subcore_barrier()
    pltpu.sync_copy(acc_sh.at[pl.ds(r0, RPT)],
                    out_hbm.at[cid, pl.ds(r0, RPT)])

    @pl.when(sid == 0)
    def _flush_tail():
        pltpu.sync_copy(acc_sh.at[pl.ds(NS * RPT, 16)],
                        out_hbm.at[cid, pl.ds(NS * RPT, 16)])


def kernel(x, edge_index, edge_weights, W0, b0, hop1, W1, b1,
           hop2, W2, b2, Wh1, bh1, Wh2, bh2):
    src = edge_index[0]
    dst = edge_index[1]
    gidx = _gidx(src.reshape(NCHUNK, B),
                 edge_weights.reshape(NCHUNK, B)).reshape(E)

    h0, s1 = _mlp_scale(x, W0, b0.reshape(1, D), hop1.reshape(1, K))
    p1 = _sc_agg(s1.reshape(K * N, D), gidx, dst)
    h1, s2 = _combine_scale(h0, p1[0], p1[1], W1, b1.reshape(1, D),
                            hop2.reshape(1, K))
    p2 = _sc_agg(s2.reshape(K * N, D), gidx, dst)
    out = _combine_head(h1, p2[0], p2[1], W2, b2.reshape(1, D),
                        Wh1, bh1.reshape(1, D), Wh2, bh2.reshape(1, C))
    return out


# speedup vs baseline: 2.8444x; 1.0239x over previous
import functools

import jax
import jax.numpy as jnp
from jax import lax
from jax.experimental import pallas as pl
from jax.experimental.pallas import tpu as pltpu
from jax.experimental.pallas import tpu_sc as plsc

N = 10000
E = 320000
D = 128
K = 5
C = 64

BR = 400
NB = N // BR
NC, NS = 2, 16
NW = NC * NS
B = 128
NCHUNK = E // B
NCW = NCHUNK // NW
NCX = NCHUNK % NW
NIH = NCW // 2
NROWS = N
RPT = 624
ZR = 80


def _softmax_row(hop_ref):
    hrow = hop_ref[...]
    m = jnp.max(hrow)
    e = jnp.exp(hrow - m)
    return e / jnp.sum(e)


def _mlp_scale_body(x_ref, w_ref, b_ref, hop_ref, h_ref, s_ref):
    h = jnp.maximum(
        jnp.dot(x_ref[...], w_ref[...], preferred_element_type=jnp.float32)
        + b_ref[...], 0.0)
    h_ref[...] = h
    w = _softmax_row(hop_ref)
    for kk in range(K):
        s_ref[kk] = h * w[0, kk]


_mlp_scale = pl.pallas_call(
    _mlp_scale_body,
    grid=(NB,),
    in_specs=[pl.BlockSpec((BR, D), lambda i: (i, 0)),
              pl.BlockSpec((D, D), lambda i: (0, 0)),
              pl.BlockSpec((1, D), lambda i: (0, 0)),
              pl.BlockSpec((1, K), lambda i: (0, 0))],
    out_specs=[pl.BlockSpec((BR, D), lambda i: (i, 0)),
               pl.BlockSpec((K, BR, D), lambda i: (0, i, 0))],
    out_shape=[jax.ShapeDtypeStruct((N, D), jnp.float32),
               jax.ShapeDtypeStruct((K, N, D), jnp.float32)],
)


def _combine_scale_body(h_ref, a0_ref, a1_ref, w_ref, b_ref, hop_ref,
                        h1_ref, s_ref):
    s = h_ref[...] + a0_ref[...] + a1_ref[...]
    h1 = jnp.maximum(
        jnp.dot(s, w_ref[...], preferred_element_type=jnp.float32)
        + b_ref[...], 0.0)
    h1_ref[...] = h1
    w = _softmax_row(hop_ref)
    for kk in range(K):
        s_ref[kk] = h1 * w[0, kk]


_combine_scale = pl.pallas_call(
    _combine_scale_body,
    grid=(NB,),
    in_specs=[pl.BlockSpec((BR, D), lambda i: (i, 0)),
              pl.BlockSpec((BR, D), lambda i: (i, 0)),
              pl.BlockSpec((BR, D), lambda i: (i, 0)),
              pl.BlockSpec((D, D), lambda i: (0, 0)),
              pl.BlockSpec((1, D), lambda i: (0, 0)),
              pl.BlockSpec((1, K), lambda i: (0, 0))],
    out_specs=[pl.BlockSpec((BR, D), lambda i: (i, 0)),
               pl.BlockSpec((K, BR, D), lambda i: (0, i, 0))],
    out_shape=[jax.ShapeDtypeStruct((N, D), jnp.float32),
               jax.ShapeDtypeStruct((K, N, D), jnp.float32)],
)


def _combine_head_body(h_ref, a0_ref, a1_ref, w_ref, b_ref,
                       w1_ref, b1_ref, w2_ref, b2_ref, o_ref):
    s = h_ref[...] + a0_ref[...] + a1_ref[...]
    h2 = jnp.maximum(
        jnp.dot(s, w_ref[...], preferred_element_type=jnp.float32)
        + b_ref[...], 0.0)
    t = jnp.maximum(
        jnp.dot(h2, w1_ref[...], preferred_element_type=jnp.float32)
        + b1_ref[...], 0.0)
    o_ref[...] = (jnp.dot(t, w2_ref[...], preferred_element_type=jnp.float32)
                  + b2_ref[...])


_combine_head = pl.pallas_call(
    _combine_head_body,
    grid=(NB,),
    in_specs=[pl.BlockSpec((BR, D), lambda i: (i, 0)),
              pl.BlockSpec((BR, D), lambda i: (i, 0)),
              pl.BlockSpec((BR, D), lambda i: (i, 0)),
              pl.BlockSpec((D, D), lambda i: (0, 0)),
              pl.BlockSpec((1, D), lambda i: (0, 0)),
              pl.BlockSpec((D, D), lambda i: (0, 0)),
              pl.BlockSpec((1, D), lambda i: (0, 0)),
              pl.BlockSpec((D, C), lambda i: (0, 0)),
              pl.BlockSpec((1, C), lambda i: (0, 0))],
    out_specs=pl.BlockSpec((BR, C), lambda i: (i, 0)),
    out_shape=jax.ShapeDtypeStruct((N, C), jnp.float32),
)


def _gidx_body(src_ref, ew_ref, o_ref):
    o_ref[...] = ew_ref[...] * N + src_ref[...]


_gidx = pl.pallas_call(
    _gidx_body,
    out_shape=jax.ShapeDtypeStruct((NCHUNK, B), jnp.int32),
)


_mesh = plsc.VectorSubcoreMesh(core_axis_name="c", subcore_axis_name="s")


@functools.partial(
    pl.kernel,
    out_type=jax.ShapeDtypeStruct((NC, N, D), jnp.float32),
    mesh=_mesh,
    scratch_types=[
        pltpu.VMEM((B,), jnp.int32),
        pltpu.VMEM((B,), jnp.int32),
        pltpu.VMEM((B,), jnp.int32),
        pltpu.VMEM((B,), jnp.int32),
        pltpu.VMEM((B, D), jnp.float32),
        pltpu.VMEM((B, D), jnp.float32),
        pltpu.VMEM((ZR, D), jnp.float32),
        pltpu.VMEM_SHARED((NROWS, D), jnp.float32),
        pltpu.SemaphoreType.DMA,
        pltpu.SemaphoreType.DMA,
    ],
)
def _sc_agg(scaled_hbm, gidx_hbm, dst_hbm, out_hbm,
            gi0_v, di0_v, gi1_v, di1_v, rows0_v, rows1_v, zbuf_v, acc_sh,
            sem0, sem1):
    cid = lax.axis_index("c")
    sid = lax.axis_index("s")
    wid = cid * NS + sid

    zv = jnp.zeros((16,), jnp.float32)

    def _zb(i, carry):
        zbuf_v[i // 8, pl.ds((i % 8) * 16, 16)] = zv
        return carry

    lax.fori_loop(0, ZR * 8, _zb, 0)
    r0 = sid * RPT
    nz = RPT // ZR
    for j in range(nz):
        pltpu.sync_copy(zbuf_v, acc_sh.at[pl.ds(r0 + j * ZR, ZR)])
    rem = RPT - nz * ZR
    pltpu.sync_copy(zbuf_v.at[pl.ds(0, rem)],
                    acc_sh.at[pl.ds(r0 + nz * ZR, rem)])

    @pl.when(sid == 0)
    def _zero_tail():
        pltpu.sync_copy(zbuf_v.at[pl.ds(0, 16)],
                        acc_sh.at[pl.ds(NS * RPT, 16)])

    plsc.subcore_barrier()

    base = (NCW * wid + jnp.minimum(wid, NCX)) * B

    def _do_chunk(off, gi_v, di_v, rows_v, sem):
        pltpu.sync_copy(gidx_hbm.at[pl.ds(off, B)], gi_v)
        pltpu.sync_copy(dst_hbm.at[pl.ds(off, B)], di_v)
        return pltpu.async_copy(scaled_hbm.at[gi_v], rows_v, sem)

    def _pair(i, carry):
        off0 = base + (2 * i) * B
        off1 = off0 + B
        i0 = pltpu.async_copy(gidx_hbm.at[pl.ds(off0, B)], gi0_v, sem0)
        i1 = pltpu.async_copy(dst_hbm.at[pl.ds(off0, B)], di0_v, sem0)
        i2 = pltpu.async_copy(gidx_hbm.at[pl.ds(off1, B)], gi1_v, sem1)
        i3 = pltpu.async_copy(dst_hbm.at[pl.ds(off1, B)], di1_v, sem1)
        i0.wait()
        i1.wait()
        g0 = pltpu.async_copy(scaled_hbm.at[gi0_v], rows0_v, sem0)
        i2.wait()
        i3.wait()
        g1 = pltpu.async_copy(scaled_hbm.at[gi1_v], rows1_v, sem1)
        g0.wait()
        pltpu.sync_copy(rows0_v, acc_sh.at[di0_v], add=True)
        g1.wait()
        pltpu.sync_copy(rows1_v, acc_sh.at[di1_v], add=True)
        return carry

    lax.fori_loop(0, NIH, _pair, 0)

    @pl.when(wid < NCX)
    def _extra_chunk():
        g = _do_chunk(base + NCW * B, gi0_v, di0_v, rows0_v, sem0)
        g.wait()
        pltpu.sync_copy(rows0_v, acc_sh.at[di0_v], add=True)

    plsc.subcore_barrier()
    pltpu.sync_copy(acc_sh.at[pl.ds(r0, RPT)],
                    out_hbm.at[cid, pl.ds(r0, RPT)])

    @pl.when(sid == 0)
    def _flush_tail():
        pltpu.sync_copy(acc_sh.at[pl.ds(NS * RPT, 16)],
                        out_hbm.at[cid, pl.ds(NS * RPT, 16)])


def kernel(x, edge_index, edge_weights, W0, b0, hop1, W1, b1,
           hop2, W2, b2, Wh1, bh1, Wh2, bh2):
    src = edge_index[0]
    dst = edge_index[1]
    gidx = _gidx(src.reshape(NCHUNK, B),
                 edge_weights.reshape(NCHUNK, B)).reshape(E)

    h0, s1 = _mlp_scale(x, W0, b0.reshape(1, D), hop1.reshape(1, K))
    p1 = _sc_agg(s1.reshape(K * N, D), gidx, dst)
    h1, s2 = _combine_scale(h0, p1[0], p1[1], W1, b1.reshape(1, D),
                            hop2.reshape(1, K))
    p2 = _sc_agg(s2.reshape(K * N, D), gidx, dst)
    out = _combine_head(h1, p2[0], p2[1], W2, b2.reshape(1, D),
                        Wh1, bh1.reshape(1, D), Wh2, bh2.reshape(1, C))
    return out


# speedup vs baseline: 2.9694x; 1.0440x over previous
import functools

import jax
import jax.numpy as jnp
from jax import lax
from jax.experimental import pallas as pl
from jax.experimental.pallas import tpu as pltpu
from jax.experimental.pallas import tpu_sc as plsc

N = 10000
E = 320000
D = 128
K = 5
C = 64

BR = 400
NB = N // BR
NC, NS = 2, 16
NW = NC * NS
B = 128
NCHUNK = E // B
NCW = NCHUNK // NW
NCX = NCHUNK % NW
NIH = NCW // 3
NROWS = N
RPT = 624


def _softmax_row(hop_ref):
    hrow = hop_ref[...]
    m = jnp.max(hrow)
    e = jnp.exp(hrow - m)
    return e / jnp.sum(e)


def _mlp_scale_body(x_ref, w_ref, b_ref, hop_ref, h_ref, s_ref):
    h = jnp.maximum(
        jnp.dot(x_ref[...], w_ref[...], preferred_element_type=jnp.float32)
        + b_ref[...], 0.0)
    h_ref[...] = h
    w = _softmax_row(hop_ref)
    for kk in range(K):
        s_ref[kk] = h * w[0, kk]


_mlp_scale = pl.pallas_call(
    _mlp_scale_body,
    grid=(NB,),
    in_specs=[pl.BlockSpec((BR, D), lambda i: (i, 0)),
              pl.BlockSpec((D, D), lambda i: (0, 0)),
              pl.BlockSpec((1, D), lambda i: (0, 0)),
              pl.BlockSpec((1, K), lambda i: (0, 0))],
    out_specs=[pl.BlockSpec((BR, D), lambda i: (i, 0)),
               pl.BlockSpec((K, BR, D), lambda i: (0, i, 0))],
    out_shape=[jax.ShapeDtypeStruct((N, D), jnp.float32),
               jax.ShapeDtypeStruct((K, N, D), jnp.float32)],
)


def _combine_scale_body(h_ref, a0_ref, a1_ref, w_ref, b_ref, hop_ref,
                        h1_ref, s_ref):
    s = h_ref[...] + a0_ref[...] + a1_ref[...]
    h1 = jnp.maximum(
        jnp.dot(s, w_ref[...], preferred_element_type=jnp.float32)
        + b_ref[...], 0.0)
    h1_ref[...] = h1
    w = _softmax_row(hop_ref)
    for kk in range(K):
        s_ref[kk] = h1 * w[0, kk]


_combine_scale = pl.pallas_call(
    _combine_scale_body,
    grid=(NB,),
    in_specs=[pl.BlockSpec((BR, D), lambda i: (i, 0)),
              pl.BlockSpec((BR, D), lambda i: (i, 0)),
              pl.BlockSpec((BR, D), lambda i: (i, 0)),
              pl.BlockSpec((D, D), lambda i: (0, 0)),
              pl.BlockSpec((1, D), lambda i: (0, 0)),
              pl.BlockSpec((1, K), lambda i: (0, 0))],
    out_specs=[pl.BlockSpec((BR, D), lambda i: (i, 0)),
               pl.BlockSpec((K, BR, D), lambda i: (0, i, 0))],
    out_shape=[jax.ShapeDtypeStruct((N, D), jnp.float32),
               jax.ShapeDtypeStruct((K, N, D), jnp.float32)],
)


def _combine_head_body(h_ref, a0_ref, a1_ref, w_ref, b_ref,
                       w1_ref, b1_ref, w2_ref, b2_ref, o_ref):
    s = h_ref[...] + a0_ref[...] + a1_ref[...]
    h2 = jnp.maximum(
        jnp.dot(s, w_ref[...], preferred_element_type=jnp.float32)
        + b_ref[...], 0.0)
    t = jnp.maximum(
        jnp.dot(h2, w1_ref[...], preferred_element_type=jnp.float32)
        + b1_ref[...], 0.0)
    o_ref[...] = (jnp.dot(t, w2_ref[...], preferred_element_type=jnp.float32)
                  + b2_ref[...])


_combine_head = pl.pallas_call(
    _combine_head_body,
    grid=(NB,),
    in_specs=[pl.BlockSpec((BR, D), lambda i: (i, 0)),
              pl.BlockSpec((BR, D), lambda i: (i, 0)),
              pl.BlockSpec((BR, D), lambda i: (i, 0)),
              pl.BlockSpec((D, D), lambda i: (0, 0)),
              pl.BlockSpec((1, D), lambda i: (0, 0)),
              pl.BlockSpec((D, D), lambda i: (0, 0)),
              pl.BlockSpec((1, D), lambda i: (0, 0)),
              pl.BlockSpec((D, C), lambda i: (0, 0)),
              pl.BlockSpec((1, C), lambda i: (0, 0))],
    out_specs=pl.BlockSpec((BR, C), lambda i: (i, 0)),
    out_shape=jax.ShapeDtypeStruct((N, C), jnp.float32),
)


def _gidx_body(src_ref, ew_ref, o_ref):
    o_ref[...] = ew_ref[...] * N + src_ref[...]


_gidx = pl.pallas_call(
    _gidx_body,
    out_shape=jax.ShapeDtypeStruct((NCHUNK, B), jnp.int32),
)


_mesh = plsc.VectorSubcoreMesh(core_axis_name="c", subcore_axis_name="s")


@functools.partial(
    pl.kernel,
    out_type=jax.ShapeDtypeStruct((NC, N, D), jnp.float32),
    mesh=_mesh,
    scratch_types=[
        pltpu.VMEM((B,), jnp.int32),
        pltpu.VMEM((B,), jnp.int32),
        pltpu.VMEM((B,), jnp.int32),
        pltpu.VMEM((B,), jnp.int32),
        pltpu.VMEM((B,), jnp.int32),
        pltpu.VMEM((B,), jnp.int32),
        pltpu.VMEM((B, D), jnp.float32),
        pltpu.VMEM((B, D), jnp.float32),
        pltpu.VMEM((B, D), jnp.float32),
        pltpu.VMEM_SHARED((NROWS, D), jnp.float32),
        pltpu.SemaphoreType.DMA,
        pltpu.SemaphoreType.DMA,
        pltpu.SemaphoreType.DMA,
    ],
)
def _sc_agg(scaled_hbm, gidx_hbm, dst_hbm, out_hbm,
            gi0_v, di0_v, gi1_v, di1_v, gi2_v, di2_v,
            rows0_v, rows1_v, rows2_v, acc_sh, sem0, sem1, sem2):
    cid = lax.axis_index("c")
    sid = lax.axis_index("s")
    wid = cid * NS + sid

    zv = jnp.zeros((16,), jnp.float32)

    def _zb(i, carry):
        rows0_v[i // 8, pl.ds((i % 8) * 16, 16)] = zv
        return carry

    lax.fori_loop(0, B * 8, _zb, 0)
    r0 = sid * RPT
    nz = RPT // B
    for j in range(nz):
        pltpu.sync_copy(rows0_v, acc_sh.at[pl.ds(r0 + j * B, B)])
    rem = RPT - nz * B
    pltpu.sync_copy(rows0_v.at[pl.ds(0, rem)],
                    acc_sh.at[pl.ds(r0 + nz * B, rem)])

    @pl.when(sid == 0)
    def _zero_tail():
        pltpu.sync_copy(rows0_v.at[pl.ds(0, 16)],
                        acc_sh.at[pl.ds(NS * RPT, 16)])

    plsc.subcore_barrier()

    base = (NCW * wid + jnp.minimum(wid, NCX)) * B

    def _triple(i, carry):
        off0 = base + (3 * i) * B
        off1 = off0 + B
        off2 = off0 + 2 * B
        ixs = []
        for off, gi_v, di_v, sem in ((off0, gi0_v, di0_v, sem0),
                                     (off1, gi1_v, di1_v, sem1),
                                     (off2, gi2_v, di2_v, sem2)):
            ixs.append((pltpu.async_copy(gidx_hbm.at[pl.ds(off, B)], gi_v,
                                         sem),
                        pltpu.async_copy(dst_hbm.at[pl.ds(off, B)], di_v,
                                         sem)))
        gs = []
        for (ia, ib), gi_v, rows_v, sem in zip(
                ixs, (gi0_v, gi1_v, gi2_v), (rows0_v, rows1_v, rows2_v),
                (sem0, sem1, sem2)):
            ia.wait()
            ib.wait()
            gs.append(pltpu.async_copy(scaled_hbm.at[gi_v], rows_v, sem))
        for g, rows_v, di_v in zip(gs, (rows0_v, rows1_v, rows2_v),
                                   (di0_v, di1_v, di2_v)):
            g.wait()
            pltpu.sync_copy(rows_v, acc_sh.at[di_v], add=True)
        return carry

    lax.fori_loop(0, NIH, _triple, 0)

    @pl.when(wid < NCX)
    def _extra_chunk():
        off = base + NCW * B
        pltpu.sync_copy(gidx_hbm.at[pl.ds(off, B)], gi0_v)
        pltpu.sync_copy(dst_hbm.at[pl.ds(off, B)], di0_v)
        pltpu.async_copy(scaled_hbm.at[gi0_v], rows0_v, sem0).wait()
        pltpu.sync_copy(rows0_v, acc_sh.at[di0_v], add=True)

    plsc.subcore_barrier()
    pltpu.sync_copy(acc_sh.at[pl.ds(r0, RPT)],
                    out_hbm.at[cid, pl.ds(r0, RPT)])

    @pl.when(sid == 0)
    def _flush_tail():
        pltpu.sync_copy(acc_sh.at[pl.ds(NS * RPT, 16)],
                        out_hbm.at[cid, pl.ds(NS * RPT, 16)])


def kernel(x, edge_index, edge_weights, W0, b0, hop1, W1, b1,
           hop2, W2, b2, Wh1, bh1, Wh2, bh2):
    src = edge_index[0]
    dst = edge_index[1]
    gidx = _gidx(src.reshape(NCHUNK, B),
                 edge_weights.reshape(NCHUNK, B)).reshape(E)

    h0, s1 = _mlp_scale(x, W0, b0.reshape(1, D), hop1.reshape(1, K))
    p1 = _sc_agg(s1.reshape(K * N, D), gidx, dst)
    h1, s2 = _combine_scale(h0, p1[0], p1[1], W1, b1.reshape(1, D),
                            hop2.reshape(1, K))
    p2 = _sc_agg(s2.reshape(K * N, D), gidx, dst)
    out = _combine_head(h1, p2[0], p2[1], W2, b2.reshape(1, D),
                        Wh1, bh1.reshape(1, D), Wh2, bh2.reshape(1, C))
    return out


# speedup vs baseline: 3.0586x; 1.0301x over previous
import functools

import jax
import jax.numpy as jnp
from jax import lax
from jax.experimental import pallas as pl
from jax.experimental.pallas import tpu as pltpu
from jax.experimental.pallas import tpu_sc as plsc

N = 10000
E = 320000
D = 128
K = 5
C = 64

BR = 400
NB = N // BR
NC, NS = 2, 16
NW = NC * NS
B = 128
NCHUNK = E // B
NCW = NCHUNK // NW
NCX = NCHUNK % NW
NIH = NCW // 3
NROWS = N
RPT = 624


def _softmax_row(hop_ref):
    hrow = hop_ref[...]
    m = jnp.max(hrow)
    e = jnp.exp(hrow - m)
    return e / jnp.sum(e)


def _mlp_scale_body(x_ref, w_ref, b_ref, hop_ref, h_ref, s_ref):
    h = jnp.maximum(
        jnp.dot(x_ref[...], w_ref[...], preferred_element_type=jnp.float32)
        + b_ref[...], 0.0)
    h_ref[...] = h
    w = _softmax_row(hop_ref)
    for kk in range(K):
        s_ref[kk] = h * w[0, kk]


_mlp_scale = pl.pallas_call(
    _mlp_scale_body,
    grid=(NB,),
    in_specs=[pl.BlockSpec((BR, D), lambda i: (i, 0)),
              pl.BlockSpec((D, D), lambda i: (0, 0)),
              pl.BlockSpec((1, D), lambda i: (0, 0)),
              pl.BlockSpec((1, K), lambda i: (0, 0))],
    out_specs=[pl.BlockSpec((BR, D), lambda i: (i, 0)),
               pl.BlockSpec((K, BR, D), lambda i: (0, i, 0))],
    out_shape=[jax.ShapeDtypeStruct((N, D), jnp.float32),
               jax.ShapeDtypeStruct((K, N, D), jnp.float32)],
)


def _combine_scale_body(h_ref, p_ref, w_ref, b_ref, hop_ref,
                        h1_ref, s_ref):
    s = h_ref[...] + p_ref[0] + p_ref[1]
    h1 = jnp.maximum(
        jnp.dot(s, w_ref[...], preferred_element_type=jnp.float32)
        + b_ref[...], 0.0)
    h1_ref[...] = h1
    w = _softmax_row(hop_ref)
    for kk in range(K):
        s_ref[kk] = h1 * w[0, kk]


_combine_scale = pl.pallas_call(
    _combine_scale_body,
    grid=(NB,),
    in_specs=[pl.BlockSpec((BR, D), lambda i: (i, 0)),
              pl.BlockSpec((NC, BR, D), lambda i: (0, i, 0)),
              pl.BlockSpec((D, D), lambda i: (0, 0)),
              pl.BlockSpec((1, D), lambda i: (0, 0)),
              pl.BlockSpec((1, K), lambda i: (0, 0))],
    out_specs=[pl.BlockSpec((BR, D), lambda i: (i, 0)),
               pl.BlockSpec((K, BR, D), lambda i: (0, i, 0))],
    out_shape=[jax.ShapeDtypeStruct((N, D), jnp.float32),
               jax.ShapeDtypeStruct((K, N, D), jnp.float32)],
)


def _combine_head_body(h_ref, p_ref, w_ref, b_ref,
                       w1_ref, b1_ref, w2_ref, b2_ref, o_ref):
    s = h_ref[...] + p_ref[0] + p_ref[1]
    h2 = jnp.maximum(
        jnp.dot(s, w_ref[...], preferred_element_type=jnp.float32)
        + b_ref[...], 0.0)
    t = jnp.maximum(
        jnp.dot(h2, w1_ref[...], preferred_element_type=jnp.float32)
        + b1_ref[...], 0.0)
    o_ref[...] = (jnp.dot(t, w2_ref[...], preferred_element_type=jnp.float32)
                  + b2_ref[...])


_combine_head = pl.pallas_call(
    _combine_head_body,
    grid=(NB,),
    in_specs=[pl.BlockSpec((BR, D), lambda i: (i, 0)),
              pl.BlockSpec((NC, BR, D), lambda i: (0, i, 0)),
              pl.BlockSpec((D, D), lambda i: (0, 0)),
              pl.BlockSpec((1, D), lambda i: (0, 0)),
              pl.BlockSpec((D, D), lambda i: (0, 0)),
              pl.BlockSpec((1, D), lambda i: (0, 0)),
              pl.BlockSpec((D, C), lambda i: (0, 0)),
              pl.BlockSpec((1, C), lambda i: (0, 0))],
    out_specs=pl.BlockSpec((BR, C), lambda i: (i, 0)),
    out_shape=jax.ShapeDtypeStruct((N, C), jnp.float32),
)


def _gidx_body(src_ref, ew_ref, o_ref):
    o_ref[...] = ew_ref[...] * N + src_ref[...]


_gidx = pl.pallas_call(
    _gidx_body,
    out_shape=jax.ShapeDtypeStruct((NCHUNK, B), jnp.int32),
)


_mesh = plsc.VectorSubcoreMesh(core_axis_name="c", subcore_axis_name="s")


@functools.partial(
    pl.kernel,
    out_type=jax.ShapeDtypeStruct((NC, N, D), jnp.float32),
    mesh=_mesh,
    scratch_types=[
        pltpu.VMEM((B,), jnp.int32),
        pltpu.VMEM((B,), jnp.int32),
        pltpu.VMEM((B,), jnp.int32),
        pltpu.VMEM((B,), jnp.int32),
        pltpu.VMEM((B,), jnp.int32),
        pltpu.VMEM((B,), jnp.int32),
        pltpu.VMEM((B, D), jnp.float32),
        pltpu.VMEM((B, D), jnp.float32),
        pltpu.VMEM((B, D), jnp.float32),
        pltpu.VMEM_SHARED((NROWS, D), jnp.float32),
        pltpu.SemaphoreType.DMA,
        pltpu.SemaphoreType.DMA,
        pltpu.SemaphoreType.DMA,
    ],
)
def _sc_agg(scaled_hbm, gidx_hbm, dst_hbm, out_hbm,
            gi0_v, di0_v, gi1_v, di1_v, gi2_v, di2_v,
            rows0_v, rows1_v, rows2_v, acc_sh, sem0, sem1, sem2):
    cid = lax.axis_index("c")
    sid = lax.axis_index("s")
    wid = cid * NS + sid

    zv = jnp.zeros((16,), jnp.float32)

    def _zb(i, carry):
        rows0_v[i // 8, pl.ds((i % 8) * 16, 16)] = zv
        return carry

    lax.fori_loop(0, B * 8, _zb, 0)
    r0 = sid * RPT
    nz = RPT // B
    for j in range(nz):
        pltpu.sync_copy(rows0_v, acc_sh.at[pl.ds(r0 + j * B, B)])
    rem = RPT - nz * B
    pltpu.sync_copy(rows0_v.at[pl.ds(0, rem)],
                    acc_sh.at[pl.ds(r0 + nz * B, rem)])

    @pl.when(sid == 0)
    def _zero_tail():
        pltpu.sync_copy(rows0_v.at[pl.ds(0, 16)],
                        acc_sh.at[pl.ds(NS * RPT, 16)])

    plsc.subcore_barrier()

    base = (NCW * wid + jnp.minimum(wid, NCX)) * B

    def _triple(i, carry):
        off0 = base + (3 * i) * B
        off1 = off0 + B
        off2 = off0 + 2 * B
        ixs = []
        for off, gi_v, di_v, sem in ((off0, gi0_v, di0_v, sem0),
                                     (off1, gi1_v, di1_v, sem1),
                                     (off2, gi2_v, di2_v, sem2)):
            ixs.append((pltpu.async_copy(gidx_hbm.at[pl.ds(off, B)], gi_v,
                                         sem),
                        pltpu.async_copy(dst_hbm.at[pl.ds(off, B)], di_v,
                                         sem)))
        gs = []
        for (ia, ib), gi_v, rows_v, sem in zip(
                ixs, (gi0_v, gi1_v, gi2_v), (rows0_v, rows1_v, rows2_v),
                (sem0, sem1, sem2)):
            ia.wait()
            ib.wait()
            gs.append(pltpu.async_copy(scaled_hbm.at[gi_v], rows_v, sem))
        for g, rows_v, di_v in zip(gs, (rows0_v, rows1_v, rows2_v),
                                   (di0_v, di1_v, di2_v)):
            g.wait()
            pltpu.sync_copy(rows_v, acc_sh.at[di_v], add=True)
        return carry

    lax.fori_loop(0, NIH, _triple, 0)

    @pl.when(wid < NCX)
    def _extra_chunk():
        off = base + NCW * B
        pltpu.sync_copy(gidx_hbm.at[pl.ds(off, B)], gi0_v)
        pltpu.sync_copy(dst_hbm.at[pl.ds(off, B)], di0_v)
        pltpu.async_copy(scaled_hbm.at[gi0_v], rows0_v, sem0).wait()
        pltpu.sync_copy(rows0_v, acc_sh.at[di0_v], add=True)

    plsc.subcore_barrier()
    pltpu.sync_copy(acc_sh.at[pl.ds(r0, RPT)],
                    out_hbm.at[cid, pl.ds(r0, RPT)])

    @pl.when(sid == 0)
    def _flush_tail():
        pltpu.sync_copy(acc_sh.at[pl.ds(NS * RPT, 16)],
                        out_hbm.at[cid, pl.ds(NS * RPT, 16)])


def kernel(x, edge_index, edge_weights, W0, b0, hop1, W1, b1,
           hop2, W2, b2, Wh1, bh1, Wh2, bh2):
    src = edge_index[0]
    dst = edge_index[1]
    gidx = _gidx(src.reshape(NCHUNK, B),
                 edge_weights.reshape(NCHUNK, B)).reshape(E)

    h0, s1 = _mlp_scale(x, W0, b0.reshape(1, D), hop1.reshape(1, K))
    p1 = _sc_agg(s1.reshape(K * N, D), gidx, dst)
    h1, s2 = _combine_scale(h0, p1, W1, b1.reshape(1, D),
                            hop2.reshape(1, K))
    p2 = _sc_agg(s2.reshape(K * N, D), gidx, dst)
    out = _combine_head(h1, p2, W2, b2.reshape(1, D),
                        Wh1, bh1.reshape(1, D), Wh2, bh2.reshape(1, C))
    return out
